# Initial kernel scaffold; baseline (speedup 1.0000x reference)
import functools

import jax
import jax.numpy as jnp
from jax import lax
from jax.experimental import pallas as pl
from jax.experimental.pallas import tpu as pltpu
from jax.experimental.pallas import tpu_sc as plsc

N = 32768
D = 1024
NC = 2
NS = 16
NW = NC * NS
CHUNK = N // NW
GROUPS = CHUNK // 16
G = 32
IDX_ROWS = CHUNK // G + 2


def _sc_body(mask_hbm, embeds_hbm, source_hbm, out_hbm,
             mask_v, pos1, src1, pos0, rows, sem):
    wid = lax.axis_index("s") * NC + lax.axis_index("c")
    base = wid * CHUNK
    iota = lax.iota(jnp.int32, 16)
    zero = jnp.zeros((16,), jnp.int32)

    pltpu.sync_copy(mask_hbm, mask_v)

    def psum_body(j, acc):
        return acc + mask_v[pl.ds(j * 16, 16)]
    acc = lax.fori_loop(0, wid * (CHUNK // 16), psum_body, zero)
    prefix = jnp.sum(acc)

    def comp_body(g, carry):
        cnt1, cnt0 = carry
        v = mask_v[pl.ds(base + g * 16, 16)]
        m = v > 0
        c1 = plsc.cumsum(v)
        c0 = plsc.cumsum(1 - v)
        d1 = cnt1 + c1 - 1
        d0 = cnt0 + c0 - 1
        pos = base + g * 16 + iota
        src = prefix + d1
        plsc.store_scatter(pos1, [d1 >> 5, d1 & (G - 1)], pos, mask=m)
        plsc.store_scatter(src1, [d1 >> 5, d1 & (G - 1)], src, mask=m)
        plsc.store_scatter(pos0, [d0 >> 5, d0 & (G - 1)], pos, mask=~m)
        n1 = plsc.all_reduce_population_count(m)
        return cnt1 + n1, cnt0 + (16 - n1)

    cnt1, cnt0 = lax.fori_loop(0, GROUPS, comp_body, (zero, zero))
    k1 = jnp.max(cnt1)
    k0 = jnp.max(cnt0)

    def pad_tail(ref2d, k):
        lrow = jnp.full((16,), (k - 1) >> 5, jnp.int32)
        lcol = jnp.full((16,), (k - 1) & (G - 1), jnp.int32)
        last = plsc.load_gather(ref2d, [lrow, lcol])
        for t in range(G // 16):
            d = k + t * 16 + iota
            plsc.store_scatter(ref2d, [d >> 5, d & (G - 1)], last)

    def run_list(tbl_hbm, gather_idx, scatter_idx, k):
        trips = (k + G - 1) // G
        def piece(j, c):
            pltpu.async_copy(tbl_hbm.at[gather_idx.at[j]], rows, sem).wait()
            pltpu.async_copy(rows, out_hbm.at[scatter_idx.at[j]], sem).wait()
            return c
        lax.fori_loop(0, trips, piece, 0)

    @pl.when(k1 > 0)
    def _():
        pad_tail(pos1, k1)
        pad_tail(src1, k1)
        run_list(source_hbm, src1, pos1, k1)

    @pl.when(k0 > 0)
    def _():
        pad_tail(pos0, k0)
        run_list(embeds_hbm, pos0, pos0, k0)


_sc_kernel = functools.partial(
    pl.kernel,
    out_type=jax.ShapeDtypeStruct((N, D), jnp.float32),
    mesh=plsc.VectorSubcoreMesh(core_axis_name="c", subcore_axis_name="s"),
    scratch_types=[
        pltpu.VMEM((N,), jnp.int32),
        pltpu.VMEM((IDX_ROWS, G), jnp.int32),
        pltpu.VMEM((IDX_ROWS, G), jnp.int32),
        pltpu.VMEM((IDX_ROWS, G), jnp.int32),
        pltpu.VMEM((G, D), jnp.float32),
        pltpu.SemaphoreType.DMA,
    ],
)(_sc_body)


@jax.jit
def kernel(inputs_embeds_row, mask_1d, source):
    mask_i32 = mask_1d.astype(jnp.int32)
    return _sc_kernel(mask_i32, inputs_embeds_row, source)


# speedup vs baseline: 1.6053x; 1.6053x over previous
import functools

import jax
import jax.numpy as jnp
from jax import lax
from jax.experimental import pallas as pl
from jax.experimental.pallas import tpu as pltpu
from jax.experimental.pallas import tpu_sc as plsc

N = 32768
D = 1024
NC = 2
NS = 16
NW = NC * NS
CHUNK = N // NW
GROUPS = CHUNK // 16
G = 32
IDX_ROWS = CHUNK // G + 2


def _sc_body(mask_hbm, embeds_hbm, source_hbm, out_hbm,
             mask_v, pos1, src1, pos0, rows, sem):
    wid = lax.axis_index("s") * NC + lax.axis_index("c")
    base = wid * CHUNK
    iota = lax.iota(jnp.int32, 16)
    zero = jnp.zeros((16,), jnp.int32)

    pltpu.sync_copy(mask_hbm, mask_v)

    def psum_body(j, acc):
        return acc + mask_v[pl.ds(j * 16, 16)]
    acc = lax.fori_loop(0, wid * (CHUNK // 16), psum_body, zero)
    prefix = jnp.sum(acc)

    def comp_body(g, carry):
        cnt1, cnt0 = carry
        v = mask_v[pl.ds(base + g * 16, 16)]
        m = v > 0
        c1 = plsc.cumsum(v)
        c0 = plsc.cumsum(1 - v)
        d1 = cnt1 + c1 - 1
        d0 = cnt0 + c0 - 1
        pos = base + g * 16 + iota
        src = prefix + d1
        plsc.store_scatter(pos1, [d1 >> 5, d1 & (G - 1)], pos, mask=m)
        plsc.store_scatter(src1, [d1 >> 5, d1 & (G - 1)], src, mask=m)
        plsc.store_scatter(pos0, [d0 >> 5, d0 & (G - 1)], pos, mask=~m)
        n1 = plsc.all_reduce_population_count(m)
        return cnt1 + n1, cnt0 + (16 - n1)

    cnt1, cnt0 = lax.fori_loop(0, GROUPS, comp_body, (zero, zero))
    k1 = jnp.max(cnt1)
    k0 = jnp.max(cnt0)

    def pad_tail(ref2d, k):
        lrow = jnp.full((16,), (k - 1) >> 5, jnp.int32)
        lcol = jnp.full((16,), (k - 1) & (G - 1), jnp.int32)
        last = plsc.load_gather(ref2d, [lrow, lcol])
        for t in range(G // 16):
            d = k + t * 16 + iota
            plsc.store_scatter(ref2d, [d >> 5, d & (G - 1)], last)

    def run_list(tbl_hbm, gather_idx, scatter_idx, k):
        trips = (k + G - 1) // G
        def piece(j, c):
            pltpu.async_copy(tbl_hbm.at[gather_idx.at[j]], rows, sem).wait()
            pltpu.async_copy(rows, out_hbm.at[scatter_idx.at[j]], sem).wait()
            return c
        lax.fori_loop(0, trips, piece, 0)

    @pl.when(k1 > 0)
    def _():
        pad_tail(pos1, k1)
        pad_tail(src1, k1)
        run_list(source_hbm, src1, pos1, k1)

    @pl.when(k0 > 0)
    def _():
        pad_tail(pos0, k0)
        run_list(embeds_hbm, pos0, pos0, k0)


_sc_kernel = functools.partial(
    pl.kernel,
    out_type=jax.ShapeDtypeStruct((N, D), jnp.float32),
    mesh=plsc.VectorSubcoreMesh(core_axis_name="c", subcore_axis_name="s"),
    scratch_types=[
        pltpu.VMEM((N,), jnp.int32),
        pltpu.VMEM((IDX_ROWS, G), jnp.int32),
        pltpu.VMEM((IDX_ROWS, G), jnp.int32),
        pltpu.VMEM((IDX_ROWS, G), jnp.int32),
        pltpu.VMEM((G, D), jnp.float32),
        pltpu.SemaphoreType.DMA,
    ],
    compiler_params=pltpu.CompilerParams(needs_layout_passes=False),
)(_sc_body)


@jax.jit
def kernel(inputs_embeds_row, mask_1d, source):
    mask_i32 = mask_1d.astype(jnp.int32)
    return _sc_kernel(mask_i32, inputs_embeds_row, source)
